# baseline (speedup 1.0000x reference)
import functools

import jax
import jax.numpy as jnp
from jax import lax
from jax.experimental import pallas as pl
from jax.experimental.pallas import tpu as pltpu
from jax.experimental.pallas import tpu_sc as plsc

N = 8192
NCHUNK = 4
CHUNK = N // NCHUNK
NW = 32
NS = 16
L = 16
G = 8

J1 = 1024
I2 = 1024
TB = 512


def _derive(px, py, pz, d_n, d_xm2, d_ym2, d_zm2):
    def body(v, _):
        sl = pl.ds(v * L, L)
        x = px[sl]
        y = py[sl]
        z = pz[sl]
        d_n[sl] = x * x + y * y + z * z
        d_xm2[sl] = -2.0 * x
        d_ym2[sl] = -2.0 * y
        d_zm2[sl] = -2.0 * z
        return 0

    lax.fori_loop(0, N // L, body, 0)


def _direction_sum(wid, s, per_w, own_x, own_y, own_z, own_n,
                   sc_n, sc_xm2, sc_ym2, sc_zm2):
    def per_block(jb, s):
        jbase = wid * per_w + jb * L
        vjx = own_x[pl.ds(jbase, L)]
        vjy = own_y[pl.ds(jbase, L)]
        vjz = own_z[pl.ds(jbase, L)]
        vjn = own_n[pl.ds(jbase, L)]
        for h in range(L // G):
            bx = [vjx[h * G + g] for g in range(G)]
            by = [vjy[h * G + g] for g in range(G)]
            bz = [vjz[h * G + g] for g in range(G)]
            bn = [vjn[h * G + g] for g in range(G)]
            for c in range(NCHUNK):

                def body(iv, maccs):
                    sl = pl.ds(iv * L, L)
                    vn = sc_n[sl]
                    vx = sc_xm2[sl]
                    vy = sc_ym2[sl]
                    vz = sc_zm2[sl]
                    out = []
                    for g in range(G):
                        w = vx * bx[g] + vy * by[g] + vz * bz[g]
                        out.append(jnp.minimum(maccs[g], vn + w))
                    return tuple(out)

                inf = jnp.full((L,), jnp.inf, dtype=jnp.float32)
                maccs = lax.fori_loop(c * (CHUNK // L), (c + 1) * (CHUNK // L),
                                      body, (inf,) * G)
                for g in range(G):
                    s = s + jnp.min(maccs[g]) + bn[g]
        return s

    return lax.fori_loop(0, per_w // L, per_block, s)


def _sc_body(x1_hbm, y1_hbm, z1_hbm, x2_hbm, y2_hbm, z2_hbm, out_hbm,
             c1x, c1y, c1z, c1n, c1xm2, c1ym2, c1zm2,
             c2x, c2y, c2z, c2n, c2xm2, c2ym2, c2zm2,
             svec):
    sid = lax.axis_index("s")
    cid = lax.axis_index("c")
    wid = cid * NS + sid

    pltpu.sync_copy(x1_hbm, c1x)
    pltpu.sync_copy(y1_hbm, c1y)
    pltpu.sync_copy(z1_hbm, c1z)
    pltpu.sync_copy(x2_hbm, c2x)
    pltpu.sync_copy(y2_hbm, c2y)
    pltpu.sync_copy(z2_hbm, c2z)

    _derive(c1x, c1y, c1z, c1n, c1xm2, c1ym2, c1zm2)
    _derive(c2x, c2y, c2z, c2n, c2xm2, c2ym2, c2zm2)

    s = jnp.float32(0.0)
    if J1:
        s = _direction_sum(wid, s, J1 // NW, c2x, c2y, c2z, c2n,
                           c1n, c1xm2, c1ym2, c1zm2)
    if I2:
        s = _direction_sum(wid, s, I2 // NW, c1x, c1y, c1z, c1n,
                           c2n, c2xm2, c2ym2, c2zm2)

    svec[...] = jnp.full((L,), s * (1.0 / L), dtype=jnp.float32)
    pltpu.sync_copy(svec, out_hbm.at[wid])


@jax.jit
def _chamfer_sc(x1, y1, z1, x2, y2, z2):
    mesh = plsc.VectorSubcoreMesh(core_axis_name="c", subcore_axis_name="s")
    vec = pltpu.VMEM((N,), jnp.float32)
    run = pl.kernel(
        _sc_body,
        out_type=jax.ShapeDtypeStruct((NW, L), jnp.float32),
        mesh=mesh,
        scratch_types=[vec] * 14 + [pltpu.VMEM((L,), jnp.float32)],
        compiler_params=pltpu.CompilerParams(needs_layout_passes=False),
    )
    return run(x1, y1, z1, x2, y2, z2)


def _tc_block_body(pom2_ref, no_ref, pst_ref, nsr_ref, out_ref):
    w = jnp.dot(pom2_ref[...], pst_ref[...],
                precision=jax.lax.Precision.HIGHEST,
                preferred_element_type=jnp.float32)
    t = w + nsr_ref[...]
    s = jnp.float32(0.0)
    for c in range(NCHUNK):
        m = jnp.min(t[:, c * CHUNK:(c + 1) * CHUNK], axis=1)
        s = s + jnp.sum(m + no_ref[:, 0])
    out_ref[...] = jnp.full((1, 8, 128), s * (1.0 / 1024), dtype=jnp.float32)


def _tc_direction(pom2, no, pst, nsr):
    nb = pom2.shape[0] // TB
    return pl.pallas_call(
        _tc_block_body,
        grid=(nb,),
        in_specs=[
            pl.BlockSpec((TB, 3), lambda b: (b, 0)),
            pl.BlockSpec((TB, 1), lambda b: (b, 0)),
            pl.BlockSpec((3, N), lambda b: (0, 0)),
            pl.BlockSpec((1, N), lambda b: (0, 0)),
        ],
        out_specs=pl.BlockSpec((1, 8, 128), lambda b: (b, 0, 0)),
        out_shape=jax.ShapeDtypeStruct((nb, 8, 128), jnp.float32),
    )(pom2, no, pst, nsr)


def kernel(output_pc, gt_pc):
    p1 = jnp.squeeze(output_pc)
    p2 = jnp.squeeze(gt_pc)

    sc_partials = _chamfer_sc(p1[:, 0], p1[:, 1], p1[:, 2],
                              p2[:, 0], p2[:, 1], p2[:, 2])
    total = jnp.sum(sc_partials)

    n1 = jnp.sum(p1 * p1, axis=1)
    n2 = jnp.sum(p2 * p2, axis=1)
    t1 = _tc_direction(-2.0 * p2[J1:], n2[J1:, None],
                       p1.T, n1[None, :])
    t2 = _tc_direction(-2.0 * p1[I2:], n1[I2:, None],
                       p2.T, n2[None, :])
    total = total + jnp.sum(t1) + jnp.sum(t2)
    return total / (NCHUNK * N)


# speedup vs baseline: 1.6986x; 1.6986x over previous
import functools

import jax
import jax.numpy as jnp
from jax import lax
from jax.experimental import pallas as pl
from jax.experimental.pallas import tpu as pltpu
from jax.experimental.pallas import tpu_sc as plsc

N = 8192
NCHUNK = 4
CHUNK = N // NCHUNK
NW = 32
NS = 16
L = 16
G = 8

J1 = 1024
I2 = 1024
TB = 512


def _derive(px, py, pz, d_n, d_xm2, d_ym2, d_zm2):
    def body(v, _):
        sl = pl.ds(v * L, L)
        x = px[sl]
        y = py[sl]
        z = pz[sl]
        d_n[sl] = x * x + y * y + z * z
        d_xm2[sl] = -2.0 * x
        d_ym2[sl] = -2.0 * y
        d_zm2[sl] = -2.0 * z
        return 0

    lax.fori_loop(0, N // L, body, 0)


def _direction_sum(wid, s, per_w, own_x, own_y, own_z, own_n,
                   sc_n, sc_xm2, sc_ym2, sc_zm2):
    def per_block(jb, s):
        jbase = wid * per_w + jb * L
        vjx = own_x[pl.ds(jbase, L)]
        vjy = own_y[pl.ds(jbase, L)]
        vjz = own_z[pl.ds(jbase, L)]
        vjn = own_n[pl.ds(jbase, L)]
        for h in range(L // G):
            bx = [vjx[h * G + g] for g in range(G)]
            by = [vjy[h * G + g] for g in range(G)]
            bz = [vjz[h * G + g] for g in range(G)]
            bn = [vjn[h * G + g] for g in range(G)]
            for c in range(NCHUNK):

                def body(iv, maccs):
                    sl = pl.ds(iv * L, L)
                    vn = sc_n[sl]
                    vx = sc_xm2[sl]
                    vy = sc_ym2[sl]
                    vz = sc_zm2[sl]
                    out = []
                    for g in range(G):
                        w = vx * bx[g] + vy * by[g] + vz * bz[g]
                        out.append(jnp.minimum(maccs[g], vn + w))
                    return tuple(out)

                inf = jnp.full((L,), jnp.inf, dtype=jnp.float32)
                maccs = lax.fori_loop(c * (CHUNK // L), (c + 1) * (CHUNK // L),
                                      body, (inf,) * G)
                for g in range(G):
                    s = s + jnp.min(maccs[g]) + bn[g]
        return s

    return lax.fori_loop(0, per_w // L, per_block, s)


def _sc_body(x1_hbm, y1_hbm, z1_hbm, x2_hbm, y2_hbm, z2_hbm, out_hbm,
             c1x, c1y, c1z, c1n, c1xm2, c1ym2, c1zm2,
             c2x, c2y, c2z, c2n, c2xm2, c2ym2, c2zm2,
             svec):
    sid = lax.axis_index("s")
    cid = lax.axis_index("c")
    wid = cid * NS + sid

    pltpu.sync_copy(x1_hbm, c1x)
    pltpu.sync_copy(y1_hbm, c1y)
    pltpu.sync_copy(z1_hbm, c1z)
    pltpu.sync_copy(x2_hbm, c2x)
    pltpu.sync_copy(y2_hbm, c2y)
    pltpu.sync_copy(z2_hbm, c2z)

    _derive(c1x, c1y, c1z, c1n, c1xm2, c1ym2, c1zm2)
    _derive(c2x, c2y, c2z, c2n, c2xm2, c2ym2, c2zm2)

    s = jnp.float32(0.0)
    if J1:
        s = _direction_sum(wid, s, J1 // NW, c2x, c2y, c2z, c2n,
                           c1n, c1xm2, c1ym2, c1zm2)
    if I2:
        s = _direction_sum(wid, s, I2 // NW, c1x, c1y, c1z, c1n,
                           c2n, c2xm2, c2ym2, c2zm2)

    svec[...] = jnp.full((L,), s * (1.0 / L), dtype=jnp.float32)
    pltpu.sync_copy(svec, out_hbm.at[wid])


@jax.jit
def _chamfer_sc(x1, y1, z1, x2, y2, z2):
    mesh = plsc.VectorSubcoreMesh(core_axis_name="c", subcore_axis_name="s")
    vec = pltpu.VMEM((N,), jnp.float32)
    run = pl.kernel(
        _sc_body,
        out_type=jax.ShapeDtypeStruct((NW, L), jnp.float32),
        mesh=mesh,
        scratch_types=[vec] * 14 + [pltpu.VMEM((L,), jnp.float32)],
        compiler_params=pltpu.CompilerParams(needs_layout_passes=False),
    )
    return run(x1, y1, z1, x2, y2, z2)


def _tc_block_body(pom2_ref, no_ref, pst_ref, nsr_ref, out_ref):
    a = pom2_ref[...]
    b = pst_ref[...]
    a_hi = a.astype(jnp.bfloat16)
    a_lo = (a - a_hi.astype(jnp.float32)).astype(jnp.bfloat16)
    b_hi = b.astype(jnp.bfloat16)
    b_lo = (b - b_hi.astype(jnp.float32)).astype(jnp.bfloat16)

    def dot(x, y):
        return jnp.dot(x, y, preferred_element_type=jnp.float32)

    w = dot(a_hi, b_hi) + dot(a_hi, b_lo) + dot(a_lo, b_hi)
    t = w + nsr_ref[...]
    s = jnp.float32(0.0)
    for c in range(NCHUNK):
        m = jnp.min(t[:, c * CHUNK:(c + 1) * CHUNK], axis=1)
        s = s + jnp.sum(m + no_ref[:, 0])
    out_ref[...] = jnp.full((1, 8, 128), s * (1.0 / 1024), dtype=jnp.float32)


def _tc_direction(pom2, no, pst, nsr):
    nb = pom2.shape[0] // TB
    return pl.pallas_call(
        _tc_block_body,
        grid=(nb,),
        in_specs=[
            pl.BlockSpec((TB, 3), lambda b: (b, 0)),
            pl.BlockSpec((TB, 1), lambda b: (b, 0)),
            pl.BlockSpec((3, N), lambda b: (0, 0)),
            pl.BlockSpec((1, N), lambda b: (0, 0)),
        ],
        out_specs=pl.BlockSpec((1, 8, 128), lambda b: (b, 0, 0)),
        out_shape=jax.ShapeDtypeStruct((nb, 8, 128), jnp.float32),
    )(pom2, no, pst, nsr)


def kernel(output_pc, gt_pc):
    p1 = jnp.squeeze(output_pc)
    p2 = jnp.squeeze(gt_pc)

    sc_partials = _chamfer_sc(p1[:, 0], p1[:, 1], p1[:, 2],
                              p2[:, 0], p2[:, 1], p2[:, 2])
    total = jnp.sum(sc_partials)

    n1 = jnp.sum(p1 * p1, axis=1)
    n2 = jnp.sum(p2 * p2, axis=1)
    t1 = _tc_direction(-2.0 * p2[J1:], n2[J1:, None],
                       p1.T, n1[None, :])
    t2 = _tc_direction(-2.0 * p1[I2:], n1[I2:, None],
                       p2.T, n2[None, :])
    total = total + jnp.sum(t1) + jnp.sum(t2)
    return total / (NCHUNK * N)


# speedup vs baseline: 3.2641x; 1.9216x over previous
import functools

import jax
import jax.numpy as jnp
from jax import lax
from jax.experimental import pallas as pl
from jax.experimental.pallas import tpu as pltpu
from jax.experimental.pallas import tpu_sc as plsc

N = 8192
NCHUNK = 4
CHUNK = N // NCHUNK
NW = 32
NS = 16
L = 16
G = 8

J1 = 1536
I2 = 1536
TB = 512


def _derive(px, py, pz, d_n, d_xm2, d_ym2, d_zm2):
    def body(v, _):
        sl = pl.ds(v * L, L)
        x = px[sl]
        y = py[sl]
        z = pz[sl]
        d_n[sl] = x * x + y * y + z * z
        d_xm2[sl] = -2.0 * x
        d_ym2[sl] = -2.0 * y
        d_zm2[sl] = -2.0 * z
        return 0

    lax.fori_loop(0, N // L, body, 0)


def _direction_sum(wid, s, per_w, own_x, own_y, own_z, own_n,
                   sc_n, sc_xm2, sc_ym2, sc_zm2):
    def per_block(jb, s):
        jbase = wid * per_w + jb * L
        vjx = own_x[pl.ds(jbase, L)]
        vjy = own_y[pl.ds(jbase, L)]
        vjz = own_z[pl.ds(jbase, L)]
        vjn = own_n[pl.ds(jbase, L)]
        for h in range(L // G):
            bx = [vjx[h * G + g] for g in range(G)]
            by = [vjy[h * G + g] for g in range(G)]
            bz = [vjz[h * G + g] for g in range(G)]
            bn = [vjn[h * G + g] for g in range(G)]
            for c in range(NCHUNK):

                def body(iv, maccs):
                    sl = pl.ds(iv * L, L)
                    vn = sc_n[sl]
                    vx = sc_xm2[sl]
                    vy = sc_ym2[sl]
                    vz = sc_zm2[sl]
                    out = []
                    for g in range(G):
                        w = vx * bx[g] + vy * by[g] + vz * bz[g]
                        out.append(jnp.minimum(maccs[g], vn + w))
                    return tuple(out)

                inf = jnp.full((L,), jnp.inf, dtype=jnp.float32)
                maccs = lax.fori_loop(c * (CHUNK // L), (c + 1) * (CHUNK // L),
                                      body, (inf,) * G)
                for g in range(G):
                    s = s + jnp.min(maccs[g]) + bn[g]
        return s

    return lax.fori_loop(0, per_w // L, per_block, s)


def _sc_body(x1_hbm, y1_hbm, z1_hbm, x2_hbm, y2_hbm, z2_hbm, out_hbm,
             c1x, c1y, c1z, c1n, c1xm2, c1ym2, c1zm2,
             c2x, c2y, c2z, c2n, c2xm2, c2ym2, c2zm2,
             svec):
    sid = lax.axis_index("s")
    cid = lax.axis_index("c")
    wid = cid * NS + sid

    pltpu.sync_copy(x1_hbm, c1x)
    pltpu.sync_copy(y1_hbm, c1y)
    pltpu.sync_copy(z1_hbm, c1z)
    pltpu.sync_copy(x2_hbm, c2x)
    pltpu.sync_copy(y2_hbm, c2y)
    pltpu.sync_copy(z2_hbm, c2z)

    _derive(c1x, c1y, c1z, c1n, c1xm2, c1ym2, c1zm2)
    _derive(c2x, c2y, c2z, c2n, c2xm2, c2ym2, c2zm2)

    s = jnp.float32(0.0)
    if J1:
        s = _direction_sum(wid, s, J1 // NW, c2x, c2y, c2z, c2n,
                           c1n, c1xm2, c1ym2, c1zm2)
    if I2:
        s = _direction_sum(wid, s, I2 // NW, c1x, c1y, c1z, c1n,
                           c2n, c2xm2, c2ym2, c2zm2)

    svec[...] = jnp.full((L,), s * (1.0 / L), dtype=jnp.float32)
    pltpu.sync_copy(svec, out_hbm.at[wid])


@jax.jit
def _chamfer_sc(x1, y1, z1, x2, y2, z2):
    mesh = plsc.VectorSubcoreMesh(core_axis_name="c", subcore_axis_name="s")
    vec = pltpu.VMEM((N,), jnp.float32)
    run = pl.kernel(
        _sc_body,
        out_type=jax.ShapeDtypeStruct((NW, L), jnp.float32),
        mesh=mesh,
        scratch_types=[vec] * 14 + [pltpu.VMEM((L,), jnp.float32)],
        compiler_params=pltpu.CompilerParams(needs_layout_passes=False),
    )
    return run(x1, y1, z1, x2, y2, z2)


KDIM = 11


def _tc_block_body(a_ref, no_ref, b_ref, out_ref):
    t = jnp.dot(a_ref[...], b_ref[...],
                preferred_element_type=jnp.float32)
    s = jnp.float32(0.0)
    for c in range(NCHUNK):
        m = jnp.min(t[:, c * CHUNK:(c + 1) * CHUNK], axis=1)
        s = s + jnp.sum(m + no_ref[:, 0])
    out_ref[...] = jnp.full((1, 8, 128), s * (1.0 / 1024), dtype=jnp.float32)


def _tc_direction(a, no, b):
    nb = a.shape[0] // TB
    return pl.pallas_call(
        _tc_block_body,
        grid=(nb,),
        in_specs=[
            pl.BlockSpec((TB, KDIM), lambda i: (i, 0)),
            pl.BlockSpec((TB, 1), lambda i: (i, 0)),
            pl.BlockSpec((KDIM, N), lambda i: (0, 0)),
        ],
        out_specs=pl.BlockSpec((1, 8, 128), lambda i: (i, 0, 0)),
        out_shape=jax.ShapeDtypeStruct((nb, 8, 128), jnp.float32),
    )(a, no, b)


def _split_bf16(x):
    hi = x.astype(jnp.bfloat16)
    lo = (x - hi.astype(jnp.float32)).astype(jnp.bfloat16)
    return hi, lo


def _pack_operands(own, n_own, scanned, n_scanned):
    ah, al = _split_bf16(-2.0 * own)
    ones = jnp.ones((own.shape[0], 1), jnp.bfloat16)
    a = jnp.concatenate([ah, ah, al, ones, ones], axis=1)
    bh, bl = _split_bf16(scanned.T)
    nh, nl = _split_bf16(n_scanned[None, :])
    b = jnp.concatenate([bh, bl, bh, nh, nl], axis=0)
    return a, n_own[:, None], b


def kernel(output_pc, gt_pc):
    p1 = jnp.squeeze(output_pc)
    p2 = jnp.squeeze(gt_pc)

    sc_partials = _chamfer_sc(p1[:, 0], p1[:, 1], p1[:, 2],
                              p2[:, 0], p2[:, 1], p2[:, 2])
    total = jnp.sum(sc_partials)

    n1 = jnp.sum(p1 * p1, axis=1)
    n2 = jnp.sum(p2 * p2, axis=1)
    t1 = _tc_direction(*_pack_operands(p2[J1:], n2[J1:], p1, n1))
    t2 = _tc_direction(*_pack_operands(p1[I2:], n1[I2:], p2, n2))
    total = total + jnp.sum(t1) + jnp.sum(t2)
    return total / (NCHUNK * N)
